# baseline (speedup 1.0000x reference)
import functools

import jax
import jax.numpy as jnp
from jax import lax
from jax.experimental import pallas as pl
from jax.experimental.pallas import tpu as pltpu
from jax.experimental.pallas import tpu_sc as plsc

N = 10000
NPAD = 10240
NC, NS = 2, 16
NW = NC * NS
K = 128
NBUF = 4
EPAD = 327680
JUNK = NPAD - 1
ROWS_PER_TILE = NPAD // NS

NB_H = EPAD // (NS * K)
NB_L = EPAD // (NW * K)
DH = 64

_MESH = plsc.VectorSubcoreMesh(core_axis_name="c", subcore_axis_name="s")
_PARAMS = pltpu.CompilerParams(use_tc_tiling_on_sc=False)


def _fill(ref, nrows, ncols, val):
    def fr(i, carry):
        for jj in range(ncols // 16):
            ref[i, pl.ds(jj * 16, 16)] = jnp.full((16,), val, jnp.float32)
        return carry

    lax.fori_loop(0, nrows, fr, 0)


def _zero_acc(zbuf, acc, s):
    def za(jj, carry):
        pltpu.sync_copy(zbuf, acc.at[pl.ds(s * ROWS_PER_TILE + jj * K, K)])
        return carry

    lax.fori_loop(0, ROWS_PER_TILE // K, za, 0)


def _sweep(z_hbm, idx_src, idx_dst, rows, acc, gsems, ssems, nb):
    ng = nb // NBUF
    for j in range(NBUF):
        pltpu.async_copy(z_hbm.at[idx_src.at[j]], rows[j], gsems[j])

    def group(g, carry):
        scat = []
        for j in range(NBUF):
            b = g * NBUF + j
            pltpu.make_async_copy(z_hbm.at[idx_src.at[b]], rows[j], gsems[j]).wait()
            scat.append(
                pltpu.async_copy(rows[j], acc.at[idx_dst.at[b]], ssems[j], add=True)
            )
        for j in range(NBUF):
            scat[j].wait()

            @pl.when(g + 1 < ng)
            def _():
                bn = (g + 1) * NBUF + j
                pltpu.async_copy(z_hbm.at[idx_src.at[bn]], rows[j], gsems[j])

        return carry

    lax.fori_loop(0, ng, group, 0)


def _make_hop(n_s, n_t):
    n_out = n_s + n_t

    @functools.partial(
        pl.kernel,
        out_type=tuple(
            jax.ShapeDtypeStruct((NC, NPAD, DH), jnp.float32) for _ in range(n_out)
        ),
        mesh=_MESH,
        scratch_types=(
            [
                pltpu.VMEM((NB_H, K), jnp.int32),
                pltpu.VMEM((NB_H, K), jnp.int32),
            ]
            + [pltpu.VMEM((K, DH), jnp.float32) for _ in range(NBUF)]
            + [pltpu.VMEM_SHARED((NPAD, DH), jnp.float32)]
            + [pltpu.SemaphoreType.DMA for _ in range(2 * NBUF)]
        ),
        compiler_params=_PARAMS,
    )
    def hop(dstS, srcS, dstT, srcT, tok, *rest):
        del tok
        zs = rest[:n_out]
        outs = rest[n_out:2 * n_out]
        idx_dst, idx_src = rest[2 * n_out:2 * n_out + 2]
        rows = rest[2 * n_out + 2:2 * n_out + 2 + NBUF]
        acc = rest[2 * n_out + 2 + NBUF]
        gsems = rest[2 * n_out + 3 + NBUF:2 * n_out + 3 + 2 * NBUF]
        ssems = rest[2 * n_out + 3 + 2 * NBUF:]
        c = lax.axis_index("c")
        s = lax.axis_index("s")

        _fill(rows[0], K, DH, 0.0)

        def stage(dst_hbm, src_hbm):
            pltpu.sync_copy(dst_hbm.at[s], idx_dst)
            pltpu.sync_copy(src_hbm.at[s], idx_src)

            def xf(b, carry):
                for jj in range(K // 16):
                    v = idx_src[b, pl.ds(jj * 16, 16)]
                    idx_src[b, pl.ds(jj * 16, 16)] = v * 2 + c
                return carry

            lax.fori_loop(0, NB_H, xf, 0)

        t = 0
        for dst_hbm, src_hbm, n_dir in ((dstS, srcS, n_s), (dstT, srcT, n_t)):
            if n_dir:
                stage(dst_hbm, src_hbm)
            for _ in range(n_dir):
                _zero_acc(rows[0], acc, s)
                plsc.subcore_barrier()
                _sweep(zs[t], idx_src, idx_dst, rows, acc, gsems, ssems, NB_H)
                plsc.subcore_barrier()
                pltpu.sync_copy(
                    acc.at[pl.ds(s * ROWS_PER_TILE, ROWS_PER_TILE)],
                    outs[t].at[c, pl.ds(s * ROWS_PER_TILE, ROWS_PER_TILE)],
                )
                if t + 1 < n_out:
                    _fill(rows[0], K, DH, 0.0)
                t += 1

    return hop


def _make_lvl():
    @functools.partial(
        pl.kernel,
        out_type=tuple(
            jax.ShapeDtypeStruct((NC, NPAD, 16), jnp.float32) for _ in range(2)
        ),
        mesh=_MESH,
        scratch_types=(
            [
                pltpu.VMEM((NB_L, K), jnp.int32),
                pltpu.VMEM((NB_L, K), jnp.int32),
            ]
            + [pltpu.VMEM((K, 16), jnp.float32) for _ in range(NBUF)]
            + [pltpu.VMEM_SHARED((NPAD, 16), jnp.float32)]
            + [pltpu.SemaphoreType.DMA for _ in range(2 * NBUF)]
        ),
        compiler_params=_PARAMS,
    )
    def lvl(dstS, srcS, dstT, srcT, tok, zS, zT, outS, outT, *rest):
        del tok
        idx_dst, idx_src = rest[:2]
        rows = rest[2:2 + NBUF]
        acc = rest[2 + NBUF]
        gsems = rest[3 + NBUF:3 + 2 * NBUF]
        ssems = rest[3 + 2 * NBUF:]
        c = lax.axis_index("c")
        s = lax.axis_index("s")
        w = c * NS + s

        _fill(rows[0], K, 16, 0.0)
        first = True
        for dst_hbm, src_hbm, z, out in ((dstS, srcS, zS, outS),
                                         (dstT, srcT, zT, outT)):
            pltpu.sync_copy(dst_hbm.at[w], idx_dst)
            pltpu.sync_copy(src_hbm.at[w], idx_src)
            if not first:
                _fill(rows[0], K, 16, 0.0)
            _zero_acc(rows[0], acc, s)
            plsc.subcore_barrier()
            _sweep(z, idx_src, idx_dst, rows, acc, gsems, ssems, NB_L)
            plsc.subcore_barrier()
            pltpu.sync_copy(
                acc.at[pl.ds(s * ROWS_PER_TILE, ROWS_PER_TILE)],
                out.at[c, pl.ds(s * ROWS_PER_TILE, ROWS_PER_TILE)],
            )
            first = False

    return lvl


_lvl16 = _make_lvl()
_hop1 = _make_hop(1, 1)


def _tc_combine(hcat, wcat, bias):
    BN = 512

    def body(h_ref, w_ref, b_ref, o_ref):
        o_ref[...] = (
            jnp.dot(h_ref[...], w_ref[...], preferred_element_type=jnp.float32)
            + b_ref[...]
        )

    return pl.pallas_call(
        body,
        grid=(NPAD // BN,),
        in_specs=[
            pl.BlockSpec((BN, 768), lambda i: (i, 0)),
            pl.BlockSpec((768, 128), lambda i: (0, 0)),
            pl.BlockSpec((1, 128), lambda i: (0, 0)),
        ],
        out_specs=pl.BlockSpec((BN, 128), lambda i: (i, 0)),
        out_shape=jax.ShapeDtypeStruct((NPAD, 128), jnp.float32),
    )(hcat, wcat, bias)


def _inv_sqrt(d):
    return jnp.where(d > 0, 1.0 / jnp.sqrt(jnp.where(d > 0, d, 1.0)), 0.0)


def _col16(*cols):
    z = [c[:, None] for c in cols]
    z.append(jnp.zeros((N, 16 - len(cols)), jnp.float32))
    return jnp.concatenate(z, axis=1)


def kernel(x, edge_index, W_sd, b_sd, W_ds, b_ds, W0, b0, W1, b1, W2, b2,
           W3, b3, alpha, beta, gama):
    row, col = edge_index[0], edge_index[1]
    pad = EPAD - row.shape[0]
    junk = jnp.full((pad,), JUNK, jnp.int32)
    zero = jnp.zeros((pad,), jnp.int32)
    rowp = jnp.concatenate([row, junk])
    colp_d = jnp.concatenate([col, junk])
    colp_s = jnp.concatenate([col, zero])
    rowp_s = jnp.concatenate([row, zero])
    idxL = (rowp.reshape(NW, NB_L, K), colp_s.reshape(NW, NB_L, K),
            colp_d.reshape(NW, NB_L, K), rowp_s.reshape(NW, NB_L, K))
    idxH = (rowp.reshape(NS, NB_H, K), colp_s.reshape(NS, NB_H, K),
            colp_d.reshape(NS, NB_H, K), rowp_s.reshape(NS, NB_H, K))

    def both(o):
        return (o[0] + o[1])[:N]

    def halves(o):
        return jnp.concatenate([o[0], o[1]], axis=1)[:N]

    def split(z):
        return z.reshape(2 * N, DH)

    zt = jnp.zeros((8,), jnp.float32)

    ones16 = jnp.ones((N, 16), jnp.float32)
    og, ig = _lvl16(*idxL, zt, ones16, ones16)
    out_deg = both(og)[:, 0]
    in_deg = both(ig)[:, 0]
    dout = _inv_sqrt(out_deg)
    din = _inv_sqrt(in_deg)

    qo, po = _lvl16(*idxL, zt, _col16(din), _col16(dout))
    q = dout * both(qo)[:, 0]
    p = din * both(po)[:, 0]

    r13o, r24o = _lvl16(*idxL, zt, _col16(din * p, din * q),
                        _col16(dout * q, dout * p))
    r13 = both(r13o)
    r24 = both(r24o)
    r1 = dout * r13[:, 0]
    r3 = dout * r13[:, 1]
    r2 = din * r24[:, 0]
    r4 = din * r24[:, 1]
    c1, c2, c3, c4 = _inv_sqrt(r1), _inv_sqrt(r2), _inv_sqrt(r3), _inv_sqrt(r4)

    u1o, u2o = _hop1(*idxH, r24o[0, 0, :8],
                     split(din[:, None] * x), split(dout[:, None] * x))
    v2o, v1o = _hop1(*idxH, u1o[0, 0, :8],
                     split((din * c2)[:, None] * x),
                     split((dout * c1)[:, None] * x))
    v3o, v4o = _hop1(*idxH, v2o[0, 0, :8],
                     split((din * c4)[:, None] * x),
                     split((dout * c3)[:, None] * x))
    U1, V2, V3 = halves(u1o), halves(v2o), halves(v3o)
    U2, V1, V4 = halves(u2o), halves(v1o), halves(v4o)

    h3o, h4o = _hop1(*idxH, v3o[0, 0, :8],
                     split((din * din)[:, None] * V1),
                     split((dout * dout)[:, None] * V2))
    h5o, h6o = _hop1(*idxH, h3o[0, 0, :8],
                     split((din * dout)[:, None] * V3),
                     split((dout * din)[:, None] * V4))
    H3c, H5c, H4c, H6c = halves(h3o), halves(h5o), halves(h4o), halves(h6o)

    H1 = dout[:, None] * U1
    H2 = din[:, None] * U2
    H3 = (c1 * dout)[:, None] * H3c
    H4 = (c2 * din)[:, None] * H4c
    H5 = (c3 * dout)[:, None] * H5c
    H6 = (c4 * din)[:, None] * H6c

    hcat = jnp.concatenate([H1, H2, H3, H4, H5, H6], axis=1)
    hcat = jnp.pad(hcat, ((0, NPAD - N), (0, 0)))
    a, b, g = alpha, beta, gama
    wcat = jnp.concatenate([
        a * W_sd.T, (1.0 - a) * W_ds.T,
        b * W0.T, (1.0 - b) * W1.T,
        g * W2.T, (1.0 - g) * W3.T,
    ], axis=0)
    bias = (a * b_sd + (1.0 - a) * b_ds + b * b0 + (1.0 - b) * b1
            + g * b2 + (1.0 - g) * b3)[None, :]

    return _tc_combine(hcat, wcat, bias)[:N]
